# baseline (speedup 1.0000x reference)
import jax
import jax.numpy as jnp
from jax.experimental import pallas as pl

B = 128
N = 32768
BLOCK_ROWS = 64


def _softmax_block(x_ref, o_ref):
    j = pl.program_id(1)

    @pl.when(j == 0)
    def _data():
        x = x_ref[...]
        m = jnp.max(x, axis=1, keepdims=True)
        e = jnp.exp(x - m)
        s = jnp.sum(e, axis=1, keepdims=True)
        o_ref[...] = e * (1.0 / s)

    @pl.when(j == 1)
    def _zero_col():
        o_ref[...] = jnp.zeros_like(o_ref)


def kernel(output_true):
    grid = (B // BLOCK_ROWS, 2)
    return pl.pallas_call(
        _softmax_block,
        grid=grid,
        in_specs=[pl.BlockSpec((BLOCK_ROWS, N), lambda i, j: (i, 0))],
        out_specs=pl.BlockSpec((BLOCK_ROWS, N), lambda i, j: (i, j)),
        out_shape=jax.ShapeDtypeStruct((B, N + 1), output_true.dtype),
    )(output_true)


# speedup vs baseline: 1.0288x; 1.0288x over previous
import jax
import jax.numpy as jnp
from jax.experimental import pallas as pl
from jax.experimental.pallas import tpu as pltpu

B = 128
N = 32768
BLOCK_ROWS = 32
GRID = B // BLOCK_ROWS


def _softmax_block(x_ref, o_hbm, scratch, zcol, sems, zsem):
    i = pl.program_id(0)
    slot = jax.lax.rem(i, 2)

    @pl.when(i == 0)
    def _zero_col():
        zcol[...] = jnp.zeros_like(zcol)
        pltpu.make_async_copy(
            zcol, o_hbm.at[:, pl.ds(N, 1)], zsem
        ).start()

    @pl.when(i >= 1)
    def _wait_prev():
        prev = jax.lax.rem(i - 1, 2)
        pltpu.make_async_copy(
            scratch.at[prev],
            o_hbm.at[pl.ds((i - 1) * BLOCK_ROWS, BLOCK_ROWS), pl.ds(0, N)],
            sems.at[prev],
        ).wait()

    x = x_ref[...]
    m = jnp.max(x, axis=1, keepdims=True)
    e = jnp.exp(x - m)
    s = jnp.sum(e, axis=1, keepdims=True)
    scratch[slot] = e * (1.0 / s)

    cp = pltpu.make_async_copy(
        scratch.at[slot],
        o_hbm.at[pl.ds(i * BLOCK_ROWS, BLOCK_ROWS), pl.ds(0, N)],
        sems.at[slot],
    )
    cp.start()

    @pl.when(i == GRID - 1)
    def _drain():
        cp.wait()
        pltpu.make_async_copy(
            zcol, o_hbm.at[:, pl.ds(N, 1)], zsem
        ).wait()


def kernel(output_true):
    return pl.pallas_call(
        _softmax_block,
        grid=(GRID,),
        in_specs=[pl.BlockSpec((BLOCK_ROWS, N), lambda i: (i, 0))],
        out_specs=pl.BlockSpec(memory_space=pl.ANY),
        out_shape=jax.ShapeDtypeStruct((B, N + 1), output_true.dtype),
        scratch_shapes=[
            pltpu.VMEM((2, BLOCK_ROWS, N), jnp.float32),
            pltpu.VMEM((B, 1), jnp.float32),
            pltpu.SemaphoreType.DMA((2,)),
            pltpu.SemaphoreType.DMA,
        ],
    )(output_true)


# speedup vs baseline: 1.1360x; 1.1042x over previous
import jax
import jax.numpy as jnp
from jax.experimental import pallas as pl
from jax.experimental.pallas import tpu as pltpu

B = 128
N = 32768
BLOCK_ROWS = 32
GRID = B // BLOCK_ROWS


def _softmax_block(x_ref, o_hbm, scratch, zcol, sems, zsem):
    i = pl.program_id(0)
    slot = jax.lax.rem(i, 2)

    @pl.when(i == 0)
    def _zero_col():
        zcol[...] = jnp.zeros_like(zcol)
        pltpu.make_async_copy(
            zcol, o_hbm.at[:, pl.ds(N, 1)], zsem
        ).start()

    @pl.when(i >= 2)
    def _wait_prev():
        pltpu.make_async_copy(
            scratch.at[slot],
            o_hbm.at[pl.ds((i - 2) * BLOCK_ROWS, BLOCK_ROWS), pl.ds(0, N)],
            sems.at[slot],
        ).wait()

    x = x_ref[...]
    m = jnp.max(x, axis=1, keepdims=True)
    e = jnp.exp(x - m)
    s = jnp.sum(e, axis=1, keepdims=True)
    scratch[slot] = e * (1.0 / s)

    cp = pltpu.make_async_copy(
        scratch.at[slot],
        o_hbm.at[pl.ds(i * BLOCK_ROWS, BLOCK_ROWS), pl.ds(0, N)],
        sems.at[slot],
    )
    cp.start()

    @pl.when(i == GRID - 1)
    def _drain():
        pltpu.make_async_copy(
            scratch.at[jax.lax.rem(i - 1, 2)],
            o_hbm.at[pl.ds((i - 1) * BLOCK_ROWS, BLOCK_ROWS), pl.ds(0, N)],
            sems.at[jax.lax.rem(i - 1, 2)],
        ).wait()
        cp.wait()
        pltpu.make_async_copy(
            zcol, o_hbm.at[:, pl.ds(N, 1)], zsem
        ).wait()


def kernel(output_true):
    return pl.pallas_call(
        _softmax_block,
        grid=(GRID,),
        in_specs=[pl.BlockSpec((BLOCK_ROWS, N), lambda i: (i, 0))],
        out_specs=pl.BlockSpec(memory_space=pl.ANY),
        out_shape=jax.ShapeDtypeStruct((B, N + 1), output_true.dtype),
        scratch_shapes=[
            pltpu.VMEM((2, BLOCK_ROWS, N), jnp.float32),
            pltpu.VMEM((B, 1), jnp.float32),
            pltpu.SemaphoreType.DMA((2,)),
            pltpu.SemaphoreType.DMA,
        ],
    )(output_true)
